# Initial kernel scaffold; baseline (speedup 1.0000x reference)
import jax
import jax.numpy as jnp
from jax.experimental import pallas as pl


def kernel(x, projector, data, labels):
    raise NotImplementedError("write your pallas kernel here")


# speedup vs baseline: 2.3364x; 2.3364x over previous
import functools

import jax
import jax.numpy as jnp
from jax import lax
from jax.experimental import pallas as pl
from jax.experimental.pallas import tpu as pltpu

K_NN = 15
QB = 128
_HI = jax.lax.Precision.HIGHEST


def _prep_body(xr_ref, p30_ref, dt_ref, qa0_ref, dat_ref):
    xr = xr_ref[...]
    xf = xr - jnp.mean(xr, axis=1, keepdims=True)
    xf = xf / jnp.sqrt(jnp.sum(xf * xf, axis=1, keepdims=True))
    proj = jnp.dot(xf, p30_ref[...],
                   preferred_element_type=jnp.float32)
    nq = jnp.sum(proj * proj, axis=1, keepdims=True)
    ci = lax.broadcasted_iota(jnp.int32, proj.shape, 1)
    d = proj.shape[1] - 2
    qa0_ref[...] = jnp.where(ci == d, 1.0,
                             jnp.where(ci == d + 1, nq, -2.0 * proj))
    dt = dt_ref[...]
    nd = jnp.sum(dt * dt, axis=0, keepdims=True)
    ri = lax.broadcasted_iota(jnp.int32, dt.shape, 0)
    dat_ref[...] = jnp.where(ri == d, nd, jnp.where(ri == d + 1, 0.0, dt))


def _main_body(qa0_ref, dat_ref, lab_ref, out_ref, z_scr, m_scr):
    p = pl.program_id(1)
    inf = jnp.float32(jnp.inf)

    @pl.when(p == 0)
    def _compute_z():
        d = qa0_ref.shape[1] - 2
        s2 = jnp.dot(qa0_ref[:, :d], dat_ref[:d, :],
                     preferred_element_type=jnp.float32)
        z_scr[...] = s2 + dat_ref[d:d + 1, :]
        m_scr[...] = jnp.full(m_scr.shape, -inf, jnp.float32)

    @pl.when((p >= 1) & (p <= K_NN))
    def _extract_next_min():
        zb = z_scr[...]
        cand = jnp.where(zb > m_scr[...], zb, inf)
        m_scr[...] = jnp.min(cand, axis=1, keepdims=True)

    @pl.when(p == K_NN + 1)
    def _finalize():
        zb = z_scr[...]
        d = qa0_ref.shape[1] - 2
        nq = qa0_ref[:, d + 1:d + 2]
        dist = jnp.sqrt(jnp.maximum(zb + nq, 1e-12))
        w = jnp.where(zb <= m_scr[...], jnp.exp(-dist), 0.0)
        res = jnp.dot(w, lab_ref[...], preferred_element_type=jnp.float32,
                      precision=_HI)
        out_ref[...] = jnp.log(res[:, :out_ref.shape[1]])


def kernel(x, projector, data, labels):
    B = x.shape[0]
    n_db, d_proj = data.shape
    n_cls = labels.shape[1]
    xr = x.reshape(B, -1)
    d_raw = xr.shape[1]
    da = d_proj + 2
    np_ = pl.cdiv(n_db, 128) * 128
    pad_rows = np_ - n_db
    lab_cols = pl.cdiv(n_cls, 8) * 8

    p30 = jnp.pad(projector[:, :d_proj], ((0, 0), (0, 2)))
    dt = jnp.concatenate(
        [data, jnp.full((pad_rows, d_proj), 1e3, jnp.float32)], axis=0)
    dt_t = jnp.pad(dt.T, ((0, 2), (0, 0)))
    lab_p = jnp.pad(labels, ((0, pad_rows), (0, lab_cols - n_cls)))

    qa0, dat = pl.pallas_call(
        _prep_body,
        out_shape=(
            jax.ShapeDtypeStruct((B, da), jnp.float32),
            jax.ShapeDtypeStruct((da, np_), jnp.float32),
        ),
    )(xr, p30, dt_t)

    nqb = B // QB
    out = pl.pallas_call(
        _main_body,
        grid=(nqb, K_NN + 2),
        in_specs=[
            pl.BlockSpec((QB, da), lambda qb, p: (qb, 0)),
            pl.BlockSpec((da, np_), lambda qb, p: (0, 0)),
            pl.BlockSpec((np_, lab_cols), lambda qb, p: (0, 0)),
        ],
        out_specs=pl.BlockSpec((QB, n_cls), lambda qb, p: (qb, 0)),
        out_shape=jax.ShapeDtypeStruct((B, n_cls), jnp.float32),
        scratch_shapes=[
            pltpu.VMEM((QB, np_), jnp.float32),
            pltpu.VMEM((QB, 1), jnp.float32),
        ],
    )(qa0, dat, lab_p)
    return out
